# baseline (speedup 1.0000x reference)
import functools

import numpy as np
import jax
import jax.numpy as jnp
from jax import lax
from jax.experimental import pallas as pl
from jax.experimental.pallas import tpu as pltpu
from jax.experimental.pallas import tpu_sc as plsc

_FACTOR = 3
_X = 24
_NEW = (_X - 1) * _FACTOR + 1
_NV = _NEW * _NEW
_NVP = 4912
_B = 4
_M = 8192
_NCORES = 2
_NSUB = 32
_SUB_PER_B = _NSUB // _B
_L = 16
_GROUP = 8
_S_SC = 3072
_PTS = _S_SC // _SUB_PER_B
_NGROUPS = _PTS // (_L * _GROUP)
_M_TC = _M - _S_SC
_T_TC = _M_TC // 1024


def _refine_matrix():
    f, x, n = _FACTOR, _X, _NEW
    P = np.zeros((n, x), np.float64)
    for c in range(x - 1):
        for k in range(f):
            P[c * f + k, c] += 1.0 - k / f
            P[c * f + k, c + 1] += k / f
    Q = np.zeros((n, x), np.float64)
    for r in range(x - 1):
        for k in range(f):
            Q[r * f + k, r] += 1.0 - k / f
            Q[r * f + k, r + 1] += k / f
    Q[n - 1, x - 1] = 1.0
    K = np.kron(Q, P)
    for rf in range(n):
        K[rf * n + (n - 1), (x - 1) * x + (x - 1)] += Q[rf, x - 1]
    Kp = np.zeros((_NVP, x * x), np.float32)
    Kp[:_NV] = K.astype(np.float32)
    return np.ascontiguousarray(Kp.T)


_KT = _refine_matrix()


def _prep_body(m_ref, kt_ref, out_ref):
    f = jnp.dot(m_ref[...], kt_ref[...], preferred_element_type=jnp.float32,
                precision=jax.lax.Precision.HIGHEST)
    for b in range(_B):
        f3 = f[3 * b:3 * b + 3, :]
        f3r = f3.astype(jnp.bfloat16).astype(jnp.float32)
        out_ref[4 * b:4 * b + 3, :] = -2.0 * f3r
        c = jnp.sum(f3 * f3, axis=0, keepdims=True)
        out_ref[4 * b + 3:4 * b + 4, :] = c


@functools.cache
def _get_chamfer_sc():
    return functools.partial(
        pl.kernel,
        mesh=plsc.VectorSubcoreMesh(core_axis_name="c", subcore_axis_name="s"),
        compiler_params=pltpu.CompilerParams(needs_layout_passes=False),
        out_type=[
            jax.ShapeDtypeStruct((_NSUB * _L,), jnp.float32),
            jax.ShapeDtypeStruct((_NSUB * _L,), jnp.float32),
        ],
        scratch_types=[
            pltpu.VMEM((_NVP,), jnp.float32),
            pltpu.VMEM((_NVP,), jnp.float32),
            pltpu.VMEM((_NVP,), jnp.float32),
            pltpu.VMEM((_NVP,), jnp.float32),
            pltpu.VMEM((_PTS,), jnp.float32),
            pltpu.VMEM((_PTS,), jnp.float32),
            pltpu.VMEM((_PTS,), jnp.float32),
            pltpu.VMEM((_L,), jnp.float32),
            pltpu.VMEM((_L,), jnp.float32),
        ],
    )(_chamfer_sc_body)


def _chamfer_sc_body(vtab_hbm, pc_hbm, num_hbm, den_hbm,
                     vx_v, vy_v, vz_v, cc_v, px_v, py_v, pz_v, num_v, den_v):
    wid = lax.axis_index("c") * (_NSUB // _NCORES) + lax.axis_index("s")
    b = wid // _SUB_PER_B
    s = wid % _SUB_PER_B
    pltpu.sync_copy(vtab_hbm.at[pl.ds((4 * b + 0) * _NVP, _NVP)], vx_v)
    pltpu.sync_copy(vtab_hbm.at[pl.ds((4 * b + 1) * _NVP, _NVP)], vy_v)
    pltpu.sync_copy(vtab_hbm.at[pl.ds((4 * b + 2) * _NVP, _NVP)], vz_v)
    pltpu.sync_copy(vtab_hbm.at[pl.ds((4 * b + 3) * _NVP, _NVP)], cc_v)
    pbase = b * (3 * _S_SC) + s * _PTS
    pltpu.sync_copy(pc_hbm.at[pl.ds(pbase + 0 * _S_SC, _PTS)], px_v)
    pltpu.sync_copy(pc_hbm.at[pl.ds(pbase + 1 * _S_SC, _PTS)], py_v)
    pltpu.sync_copy(pc_hbm.at[pl.ds(pbase + 2 * _S_SC, _PTS)], pz_v)

    def round_bf16(x):
        u = plsc.bitcast(x, jnp.uint32)
        lsb = (u >> jnp.uint32(16)) & jnp.uint32(1)
        u = u + jnp.uint32(0x7FFF) + lsb
        u = u & jnp.uint32(0xFFFF0000)
        return plsc.bitcast(u, jnp.float32)

    zero = jnp.zeros((_L,), jnp.float32)
    num_lanes = zero
    den_lanes = zero
    one = jnp.full((_L,), 1.0, jnp.float32)
    for g in range(_NGROUPS):
        base = g * _GROUP * _L
        pxs = [px_v[pl.ds(base + i * _L, _L)] for i in range(_GROUP)]
        pys = [py_v[pl.ds(base + i * _L, _L)] for i in range(_GROUP)]
        pzs = [pz_v[pl.ds(base + i * _L, _L)] for i in range(_GROUP)]
        pxr = [round_bf16(p) for p in pxs]
        pyr = [round_bf16(p) for p in pys]
        pzr = [round_bf16(p) for p in pzs]
        ms = []
        for i in range(_GROUP):
            px, py, pz = pxs[i], pys[i], pzs[i]
            pp = px * px + py * py + pz * pz
            keep = jnp.logical_not(
                jnp.logical_and(jnp.logical_and(px == 0.0, py == 0.0),
                                pz == 0.0))
            m = jnp.where(keep, one, zero)
            ms.append(m)
            den_lanes = den_lanes + m
            num_lanes = num_lanes + pp * m
        init = tuple(jnp.full((_L,), 3e38, jnp.float32)
                     for _ in range(_GROUP))

        def body(j, accs, pxs=pxr, pys=pyr, pzs=pzr):
            idx = jnp.full((_L,), j, dtype=jnp.int32)
            vx = plsc.load_gather(vx_v, [idx])
            vy = plsc.load_gather(vy_v, [idx])
            vz = plsc.load_gather(vz_v, [idx])
            cc = plsc.load_gather(cc_v, [idx])
            return tuple(
                jnp.minimum(a, cc + vx * px + vy * py + vz * pz)
                for a, px, py, pz in zip(accs, pxs, pys, pzs))

        accs = lax.fori_loop(0, _NV, body, init)
        for i in range(_GROUP):
            num_lanes = num_lanes + accs[i] * ms[i]

    num_v[...] = num_lanes
    den_v[...] = den_lanes
    pltpu.sync_copy(num_v, num_hbm.at[pl.ds(wid * _L, _L)])
    pltpu.sync_copy(den_v, den_hbm.at[pl.ds(wid * _L, _L)])


def _tc_chamfer_body(vtab_ref, pc_ref, num_ref, den_ref, scr_ref):
    rnd = lambda v: v.astype(jnp.bfloat16).astype(jnp.float32)
    for b in range(_B):
        for c in range(3):
            for t in range(_T_TC):
                scr_ref[c, t] = rnd(pc_ref[b, c, t])

        def body(j, accs):
            vx = vtab_ref[4 * b + 0, j]
            vy = vtab_ref[4 * b + 1, j]
            vz = vtab_ref[4 * b + 2, j]
            cc = vtab_ref[4 * b + 3, j]
            return tuple(
                jnp.minimum(a, cc + vx * scr_ref[0, t] + vy * scr_ref[1, t]
                            + vz * scr_ref[2, t])
                for t, a in enumerate(accs))

        init = tuple(jnp.full((8, 128), 3e38, jnp.float32)
                     for _ in range(_T_TC))
        accs = lax.fori_loop(0, _NV, body, init, unroll=4)

        num = jnp.zeros((8, 128), jnp.float32)
        den = jnp.zeros((8, 128), jnp.float32)
        for t in range(_T_TC):
            px = pc_ref[b, 0, t]
            py = pc_ref[b, 1, t]
            pz = pc_ref[b, 2, t]
            pp = px * px + py * py + pz * pz
            keep = jnp.logical_not((px == 0.0) & (py == 0.0) & (pz == 0.0))
            m = jnp.where(keep, 1.0, 0.0)
            den = den + m
            num = num + (pp + accs[t]) * m
        num_ref[b] = num
        den_ref[b] = den


def _combine_body(nsc_ref, dsc_ref, ntc_ref, dtc_ref, out_ref):
    n = (jnp.sum(nsc_ref[...], axis=1, keepdims=True)
         + jnp.sum(ntc_ref[...], axis=1, keepdims=True))
    d = (jnp.sum(dsc_ref[...], axis=1, keepdims=True)
         + jnp.sum(dtc_ref[...], axis=1, keepdims=True))
    r = n / d
    out_ref[...] = jnp.sum(r, axis=0, keepdims=True) * (1.0 / _B)


def kernel(vertices, pc):
    m_flat = vertices[:, :, :, -1, :].reshape(3 * _B, _X * _X)
    kt = jnp.asarray(_KT)
    vtab = pl.pallas_call(
        _prep_body,
        out_shape=jax.ShapeDtypeStruct((4 * _B, _NVP), jnp.float32),
    )(m_flat, kt)
    pc_sc = pc[:, :, :_S_SC]
    num, den = _get_chamfer_sc()(vtab.reshape(-1), pc_sc.reshape(-1))
    pc_tc = pc[:, :, _S_SC:].reshape(_B, 3, _T_TC, 8, 128)
    num_tc, den_tc = pl.pallas_call(
        _tc_chamfer_body,
        in_specs=[
            pl.BlockSpec(memory_space=pltpu.SMEM),
            pl.BlockSpec(memory_space=pltpu.VMEM),
        ],
        out_shape=[
            jax.ShapeDtypeStruct((_B, 8, 128), jnp.float32),
            jax.ShapeDtypeStruct((_B, 8, 128), jnp.float32),
        ],
        scratch_shapes=[pltpu.VMEM((3, _T_TC, 8, 128), jnp.float32)],
    )(vtab, pc_tc)
    out = pl.pallas_call(
        _combine_body,
        out_shape=jax.ShapeDtypeStruct((1, 1), jnp.float32),
    )(num.reshape(_B, _SUB_PER_B * _L), den.reshape(_B, _SUB_PER_B * _L),
      num_tc.reshape(_B, 8 * 128), den_tc.reshape(_B, 8 * 128))
    return out.reshape(())


# speedup vs baseline: 1.0011x; 1.0011x over previous
import functools

import numpy as np
import jax
import jax.numpy as jnp
from jax import lax
from jax.experimental import pallas as pl
from jax.experimental.pallas import tpu as pltpu
from jax.experimental.pallas import tpu_sc as plsc

_FACTOR = 3
_X = 24
_NEW = (_X - 1) * _FACTOR + 1
_NV = _NEW * _NEW
_NVP = 4912
_B = 4
_M = 8192
_NCORES = 2
_NSUB = 32
_SUB_PER_B = _NSUB // _B
_L = 16
_GROUP = 8
_S_SC = 3072
_PTS = _S_SC // _SUB_PER_B
_NGROUPS = _PTS // (_L * _GROUP)
_M_TC = _M - _S_SC
_T_TC = _M_TC // 1024


def _refine_matrix():
    f, x, n = _FACTOR, _X, _NEW
    P = np.zeros((n, x), np.float64)
    for c in range(x - 1):
        for k in range(f):
            P[c * f + k, c] += 1.0 - k / f
            P[c * f + k, c + 1] += k / f
    Q = np.zeros((n, x), np.float64)
    for r in range(x - 1):
        for k in range(f):
            Q[r * f + k, r] += 1.0 - k / f
            Q[r * f + k, r + 1] += k / f
    Q[n - 1, x - 1] = 1.0
    K = np.kron(Q, P)
    for rf in range(n):
        K[rf * n + (n - 1), (x - 1) * x + (x - 1)] += Q[rf, x - 1]
    Kp = np.zeros((_NVP, x * x), np.float32)
    Kp[:_NV] = K.astype(np.float32)
    return np.ascontiguousarray(Kp.T)


_KT = _refine_matrix()


def _prep_body(m_ref, kt_ref, out_ref):
    f = jnp.dot(m_ref[...], kt_ref[...], preferred_element_type=jnp.float32,
                precision=jax.lax.Precision.HIGHEST)
    for b in range(_B):
        f3 = f[3 * b:3 * b + 3, :]
        f3r = f3.astype(jnp.bfloat16).astype(jnp.float32)
        out_ref[4 * b:4 * b + 3, :] = -2.0 * f3r
        c = jnp.sum(f3 * f3, axis=0, keepdims=True)
        out_ref[4 * b + 3:4 * b + 4, :] = c


@functools.cache
def _get_chamfer_sc():
    return functools.partial(
        pl.kernel,
        mesh=plsc.VectorSubcoreMesh(core_axis_name="c", subcore_axis_name="s"),
        compiler_params=pltpu.CompilerParams(needs_layout_passes=False),
        out_type=[
            jax.ShapeDtypeStruct((_NSUB * _L,), jnp.float32),
            jax.ShapeDtypeStruct((_NSUB * _L,), jnp.float32),
        ],
        scratch_types=[
            pltpu.VMEM((_NVP,), jnp.float32),
            pltpu.VMEM((_NVP,), jnp.float32),
            pltpu.VMEM((_NVP,), jnp.float32),
            pltpu.VMEM((_NVP,), jnp.float32),
            pltpu.VMEM((_PTS,), jnp.float32),
            pltpu.VMEM((_PTS,), jnp.float32),
            pltpu.VMEM((_PTS,), jnp.float32),
            pltpu.VMEM((_L,), jnp.float32),
            pltpu.VMEM((_L,), jnp.float32),
        ],
    )(_chamfer_sc_body)


def _chamfer_sc_body(vtab_hbm, pc_hbm, num_hbm, den_hbm,
                     vx_v, vy_v, vz_v, cc_v, px_v, py_v, pz_v, num_v, den_v):
    wid = lax.axis_index("c") * (_NSUB // _NCORES) + lax.axis_index("s")
    b = wid // _SUB_PER_B
    s = wid % _SUB_PER_B
    pltpu.sync_copy(vtab_hbm.at[pl.ds((4 * b + 0) * _NVP, _NVP)], vx_v)
    pltpu.sync_copy(vtab_hbm.at[pl.ds((4 * b + 1) * _NVP, _NVP)], vy_v)
    pltpu.sync_copy(vtab_hbm.at[pl.ds((4 * b + 2) * _NVP, _NVP)], vz_v)
    pltpu.sync_copy(vtab_hbm.at[pl.ds((4 * b + 3) * _NVP, _NVP)], cc_v)
    pbase = b * (3 * _S_SC) + s * _PTS
    pltpu.sync_copy(pc_hbm.at[pl.ds(pbase + 0 * _S_SC, _PTS)], px_v)
    pltpu.sync_copy(pc_hbm.at[pl.ds(pbase + 1 * _S_SC, _PTS)], py_v)
    pltpu.sync_copy(pc_hbm.at[pl.ds(pbase + 2 * _S_SC, _PTS)], pz_v)

    def round_bf16(x):
        u = plsc.bitcast(x, jnp.uint32)
        lsb = (u >> jnp.uint32(16)) & jnp.uint32(1)
        u = u + jnp.uint32(0x7FFF) + lsb
        u = u & jnp.uint32(0xFFFF0000)
        return plsc.bitcast(u, jnp.float32)

    zero = jnp.zeros((_L,), jnp.float32)
    num_lanes = zero
    den_lanes = zero
    one = jnp.full((_L,), 1.0, jnp.float32)
    for g in range(_NGROUPS):
        base = g * _GROUP * _L
        pxs = [px_v[pl.ds(base + i * _L, _L)] for i in range(_GROUP)]
        pys = [py_v[pl.ds(base + i * _L, _L)] for i in range(_GROUP)]
        pzs = [pz_v[pl.ds(base + i * _L, _L)] for i in range(_GROUP)]
        pxr = [round_bf16(p) for p in pxs]
        pyr = [round_bf16(p) for p in pys]
        pzr = [round_bf16(p) for p in pzs]
        ms = []
        for i in range(_GROUP):
            px, py, pz = pxs[i], pys[i], pzs[i]
            pp = px * px + py * py + pz * pz
            keep = jnp.logical_not(
                jnp.logical_and(jnp.logical_and(px == 0.0, py == 0.0),
                                pz == 0.0))
            m = jnp.where(keep, one, zero)
            ms.append(m)
            den_lanes = den_lanes + m
            num_lanes = num_lanes + pp * m
        init = tuple(jnp.full((_L,), 3e38, jnp.float32)
                     for _ in range(_GROUP))

        def body(j, accs, pxs=pxr, pys=pyr, pzs=pzr):
            idx = jnp.full((_L,), j, dtype=jnp.int32)
            vx = plsc.load_gather(vx_v, [idx])
            vy = plsc.load_gather(vy_v, [idx])
            vz = plsc.load_gather(vz_v, [idx])
            cc = plsc.load_gather(cc_v, [idx])
            return tuple(
                jnp.minimum(a, cc + vx * px + vy * py + vz * pz)
                for a, px, py, pz in zip(accs, pxs, pys, pzs))

        accs = lax.fori_loop(0, _NV, body, init)
        for i in range(_GROUP):
            num_lanes = num_lanes + accs[i] * ms[i]

    num_v[...] = num_lanes
    den_v[...] = den_lanes
    pltpu.sync_copy(num_v, num_hbm.at[pl.ds(wid * _L, _L)])
    pltpu.sync_copy(den_v, den_hbm.at[pl.ds(wid * _L, _L)])


def _tc_chamfer_body(vtab_ref, pc_ref, num_ref, den_ref, scr_ref):
    rnd = lambda v: v.astype(jnp.bfloat16).astype(jnp.float32)
    for b in range(_B):
        for c in range(3):
            for t in range(_T_TC):
                scr_ref[c, t] = rnd(pc_ref[b, c, t])

        def body(j, accs):
            vx = vtab_ref[4 * b + 0, j]
            vy = vtab_ref[4 * b + 1, j]
            vz = vtab_ref[4 * b + 2, j]
            cc = vtab_ref[4 * b + 3, j]
            return tuple(
                jnp.minimum(a, cc + vx * scr_ref[0, t] + vy * scr_ref[1, t]
                            + vz * scr_ref[2, t])
                for t, a in enumerate(accs))

        init = tuple(jnp.full((8, 128), 3e38, jnp.float32)
                     for _ in range(_T_TC))
        accs = lax.fori_loop(0, _NV, body, init, unroll=7)

        num = jnp.zeros((8, 128), jnp.float32)
        den = jnp.zeros((8, 128), jnp.float32)
        for t in range(_T_TC):
            px = pc_ref[b, 0, t]
            py = pc_ref[b, 1, t]
            pz = pc_ref[b, 2, t]
            pp = px * px + py * py + pz * pz
            keep = jnp.logical_not((px == 0.0) & (py == 0.0) & (pz == 0.0))
            m = jnp.where(keep, 1.0, 0.0)
            den = den + m
            num = num + (pp + accs[t]) * m
        num_ref[b] = num
        den_ref[b] = den


def _combine_body(nsc_ref, dsc_ref, ntc_ref, dtc_ref, out_ref):
    n = (jnp.sum(nsc_ref[...], axis=1, keepdims=True)
         + jnp.sum(ntc_ref[...], axis=1, keepdims=True))
    d = (jnp.sum(dsc_ref[...], axis=1, keepdims=True)
         + jnp.sum(dtc_ref[...], axis=1, keepdims=True))
    r = n / d
    out_ref[...] = jnp.sum(r, axis=0, keepdims=True) * (1.0 / _B)


def kernel(vertices, pc):
    m_flat = vertices[:, :, :, -1, :].reshape(3 * _B, _X * _X)
    kt = jnp.asarray(_KT)
    vtab = pl.pallas_call(
        _prep_body,
        out_shape=jax.ShapeDtypeStruct((4 * _B, _NVP), jnp.float32),
    )(m_flat, kt)
    pc_sc = pc[:, :, :_S_SC]
    num, den = _get_chamfer_sc()(vtab.reshape(-1), pc_sc.reshape(-1))
    pc_tc = pc[:, :, _S_SC:].reshape(_B, 3, _T_TC, 8, 128)
    num_tc, den_tc = pl.pallas_call(
        _tc_chamfer_body,
        in_specs=[
            pl.BlockSpec(memory_space=pltpu.SMEM),
            pl.BlockSpec(memory_space=pltpu.VMEM),
        ],
        out_shape=[
            jax.ShapeDtypeStruct((_B, 8, 128), jnp.float32),
            jax.ShapeDtypeStruct((_B, 8, 128), jnp.float32),
        ],
        scratch_shapes=[pltpu.VMEM((3, _T_TC, 8, 128), jnp.float32)],
    )(vtab, pc_tc)
    out = pl.pallas_call(
        _combine_body,
        out_shape=jax.ShapeDtypeStruct((1, 1), jnp.float32),
    )(num.reshape(_B, _SUB_PER_B * _L), den.reshape(_B, _SUB_PER_B * _L),
      num_tc.reshape(_B, 8 * 128), den_tc.reshape(_B, 8 * 128))
    return out.reshape(())


# speedup vs baseline: 1.0746x; 1.0733x over previous
import functools

import numpy as np
import jax
import jax.numpy as jnp
from jax import lax
from jax.experimental import pallas as pl
from jax.experimental.pallas import tpu as pltpu
from jax.experimental.pallas import tpu_sc as plsc

_FACTOR = 3
_X = 24
_NEW = (_X - 1) * _FACTOR + 1
_NV = _NEW * _NEW
_NVP = 4912
_B = 4
_M = 8192
_NCORES = 2
_NSUB = 32
_SUB_PER_B = _NSUB // _B
_L = 16
_GROUP = 8
_S_SC = 2048
_PTS = _S_SC // _SUB_PER_B
_NGROUPS = _PTS // (_L * _GROUP)
_M_TC = _M - _S_SC
_T_TC = _M_TC // 1024


def _refine_matrix():
    f, x, n = _FACTOR, _X, _NEW
    P = np.zeros((n, x), np.float64)
    for c in range(x - 1):
        for k in range(f):
            P[c * f + k, c] += 1.0 - k / f
            P[c * f + k, c + 1] += k / f
    Q = np.zeros((n, x), np.float64)
    for r in range(x - 1):
        for k in range(f):
            Q[r * f + k, r] += 1.0 - k / f
            Q[r * f + k, r + 1] += k / f
    Q[n - 1, x - 1] = 1.0
    K = np.kron(Q, P)
    for rf in range(n):
        K[rf * n + (n - 1), (x - 1) * x + (x - 1)] += Q[rf, x - 1]
    Kp = np.zeros((_NVP, x * x), np.float32)
    Kp[:_NV] = K.astype(np.float32)
    return np.ascontiguousarray(Kp.T)


_KT = _refine_matrix()


def _prep_body(m_ref, kt_ref, out_ref):
    f = jnp.dot(m_ref[...], kt_ref[...], preferred_element_type=jnp.float32,
                precision=jax.lax.Precision.HIGHEST)
    for b in range(_B):
        f3 = f[3 * b:3 * b + 3, :]
        f3r = f3.astype(jnp.bfloat16).astype(jnp.float32)
        out_ref[4 * b:4 * b + 3, :] = -2.0 * f3r
        c = jnp.sum(f3 * f3, axis=0, keepdims=True)
        out_ref[4 * b + 3:4 * b + 4, :] = c


@functools.cache
def _get_chamfer_sc():
    return functools.partial(
        pl.kernel,
        mesh=plsc.VectorSubcoreMesh(core_axis_name="c", subcore_axis_name="s"),
        compiler_params=pltpu.CompilerParams(needs_layout_passes=False),
        out_type=[
            jax.ShapeDtypeStruct((_NSUB * _L,), jnp.float32),
            jax.ShapeDtypeStruct((_NSUB * _L,), jnp.float32),
        ],
        scratch_types=[
            pltpu.VMEM((_NVP,), jnp.float32),
            pltpu.VMEM((_NVP,), jnp.float32),
            pltpu.VMEM((_NVP,), jnp.float32),
            pltpu.VMEM((_NVP,), jnp.float32),
            pltpu.VMEM((_PTS,), jnp.float32),
            pltpu.VMEM((_PTS,), jnp.float32),
            pltpu.VMEM((_PTS,), jnp.float32),
            pltpu.VMEM((_L,), jnp.float32),
            pltpu.VMEM((_L,), jnp.float32),
        ],
    )(_chamfer_sc_body)


def _chamfer_sc_body(vtab_hbm, pc_hbm, num_hbm, den_hbm,
                     vx_v, vy_v, vz_v, cc_v, px_v, py_v, pz_v, num_v, den_v):
    wid = lax.axis_index("c") * (_NSUB // _NCORES) + lax.axis_index("s")
    b = wid // _SUB_PER_B
    s = wid % _SUB_PER_B
    pltpu.sync_copy(vtab_hbm.at[pl.ds((4 * b + 0) * _NVP, _NVP)], vx_v)
    pltpu.sync_copy(vtab_hbm.at[pl.ds((4 * b + 1) * _NVP, _NVP)], vy_v)
    pltpu.sync_copy(vtab_hbm.at[pl.ds((4 * b + 2) * _NVP, _NVP)], vz_v)
    pltpu.sync_copy(vtab_hbm.at[pl.ds((4 * b + 3) * _NVP, _NVP)], cc_v)
    pbase = b * (3 * _S_SC) + s * _PTS
    pltpu.sync_copy(pc_hbm.at[pl.ds(pbase + 0 * _S_SC, _PTS)], px_v)
    pltpu.sync_copy(pc_hbm.at[pl.ds(pbase + 1 * _S_SC, _PTS)], py_v)
    pltpu.sync_copy(pc_hbm.at[pl.ds(pbase + 2 * _S_SC, _PTS)], pz_v)

    def round_bf16(x):
        u = plsc.bitcast(x, jnp.uint32)
        lsb = (u >> jnp.uint32(16)) & jnp.uint32(1)
        u = u + jnp.uint32(0x7FFF) + lsb
        u = u & jnp.uint32(0xFFFF0000)
        return plsc.bitcast(u, jnp.float32)

    zero = jnp.zeros((_L,), jnp.float32)
    num_lanes = zero
    den_lanes = zero
    one = jnp.full((_L,), 1.0, jnp.float32)
    for g in range(_NGROUPS):
        base = g * _GROUP * _L
        pxs = [px_v[pl.ds(base + i * _L, _L)] for i in range(_GROUP)]
        pys = [py_v[pl.ds(base + i * _L, _L)] for i in range(_GROUP)]
        pzs = [pz_v[pl.ds(base + i * _L, _L)] for i in range(_GROUP)]
        pxr = [round_bf16(p) for p in pxs]
        pyr = [round_bf16(p) for p in pys]
        pzr = [round_bf16(p) for p in pzs]
        ms = []
        for i in range(_GROUP):
            px, py, pz = pxs[i], pys[i], pzs[i]
            pp = px * px + py * py + pz * pz
            keep = jnp.logical_not(
                jnp.logical_and(jnp.logical_and(px == 0.0, py == 0.0),
                                pz == 0.0))
            m = jnp.where(keep, one, zero)
            ms.append(m)
            den_lanes = den_lanes + m
            num_lanes = num_lanes + pp * m
        init = tuple(jnp.full((_L,), 3e38, jnp.float32)
                     for _ in range(_GROUP))

        def body(j, accs, pxs=pxr, pys=pyr, pzs=pzr):
            idx = jnp.full((_L,), j, dtype=jnp.int32)
            vx = plsc.load_gather(vx_v, [idx])
            vy = plsc.load_gather(vy_v, [idx])
            vz = plsc.load_gather(vz_v, [idx])
            cc = plsc.load_gather(cc_v, [idx])
            return tuple(
                jnp.minimum(a, cc + vx * px + vy * py + vz * pz)
                for a, px, py, pz in zip(accs, pxs, pys, pzs))

        accs = lax.fori_loop(0, _NV, body, init)
        for i in range(_GROUP):
            num_lanes = num_lanes + accs[i] * ms[i]

    num_v[...] = num_lanes
    den_v[...] = den_lanes
    pltpu.sync_copy(num_v, num_hbm.at[pl.ds(wid * _L, _L)])
    pltpu.sync_copy(den_v, den_hbm.at[pl.ds(wid * _L, _L)])


def _tc_chamfer_body(vtab_ref, pc_ref, num_ref, den_ref, scr_ref):
    rnd = lambda v: v.astype(jnp.bfloat16).astype(jnp.float32)
    for b in range(_B):
        for c in range(3):
            for t in range(_T_TC):
                scr_ref[c, t] = rnd(pc_ref[b, c, t])

        def body(j, accs):
            vx = vtab_ref[4 * b + 0, j]
            vy = vtab_ref[4 * b + 1, j]
            vz = vtab_ref[4 * b + 2, j]
            cc = vtab_ref[4 * b + 3, j]
            return tuple(
                jnp.minimum(a, cc + vx * scr_ref[0, t] + vy * scr_ref[1, t]
                            + vz * scr_ref[2, t])
                for t, a in enumerate(accs))

        init = tuple(jnp.full((8, 128), 3e38, jnp.float32)
                     for _ in range(_T_TC))
        accs = lax.fori_loop(0, _NV, body, init, unroll=7)

        num = jnp.zeros((8, 128), jnp.float32)
        den = jnp.zeros((8, 128), jnp.float32)
        for t in range(_T_TC):
            px = pc_ref[b, 0, t]
            py = pc_ref[b, 1, t]
            pz = pc_ref[b, 2, t]
            pp = px * px + py * py + pz * pz
            keep = jnp.logical_not((px == 0.0) & (py == 0.0) & (pz == 0.0))
            m = jnp.where(keep, 1.0, 0.0)
            den = den + m
            num = num + (pp + accs[t]) * m
        num_ref[b] = num
        den_ref[b] = den


def _combine_body(nsc_ref, dsc_ref, ntc_ref, dtc_ref, out_ref):
    n = (jnp.sum(nsc_ref[...], axis=1, keepdims=True)
         + jnp.sum(ntc_ref[...], axis=1, keepdims=True))
    d = (jnp.sum(dsc_ref[...], axis=1, keepdims=True)
         + jnp.sum(dtc_ref[...], axis=1, keepdims=True))
    r = n / d
    out_ref[...] = jnp.sum(r, axis=0, keepdims=True) * (1.0 / _B)


def kernel(vertices, pc):
    m_flat = vertices[:, :, :, -1, :].reshape(3 * _B, _X * _X)
    kt = jnp.asarray(_KT)
    vtab = pl.pallas_call(
        _prep_body,
        out_shape=jax.ShapeDtypeStruct((4 * _B, _NVP), jnp.float32),
    )(m_flat, kt)
    pc_sc = pc[:, :, :_S_SC]
    num, den = _get_chamfer_sc()(vtab.reshape(-1), pc_sc.reshape(-1))
    pc_tc = pc[:, :, _S_SC:].reshape(_B, 3, _T_TC, 8, 128)
    num_tc, den_tc = pl.pallas_call(
        _tc_chamfer_body,
        in_specs=[
            pl.BlockSpec(memory_space=pltpu.SMEM),
            pl.BlockSpec(memory_space=pltpu.VMEM),
        ],
        out_shape=[
            jax.ShapeDtypeStruct((_B, 8, 128), jnp.float32),
            jax.ShapeDtypeStruct((_B, 8, 128), jnp.float32),
        ],
        scratch_shapes=[pltpu.VMEM((3, _T_TC, 8, 128), jnp.float32)],
    )(vtab, pc_tc)
    out = pl.pallas_call(
        _combine_body,
        out_shape=jax.ShapeDtypeStruct((1, 1), jnp.float32),
    )(num.reshape(_B, _SUB_PER_B * _L), den.reshape(_B, _SUB_PER_B * _L),
      num_tc.reshape(_B, 8 * 128), den_tc.reshape(_B, 8 * 128))
    return out.reshape(())
